# baseline (speedup 1.0000x reference)
import functools

import jax
import jax.numpy as jnp
from jax import lax
from jax.experimental import pallas as pl
from jax.experimental.pallas import tpu as pltpu
from jax.experimental.pallas import tpu_sc as plsc

NUM_EMBEDDINGS = 1000000
EMBED_SIZE = 64
BATCH = 16384
LANES = 16
TILE_W = 128
LAST_J = NUM_EMBEDDINGS // TILE_W
LAST_W = NUM_EMBEDDINGS - LAST_J * TILE_W
NUM_J = LAST_J + 1
J_PER_TEC = 245
WT = 4
NWIN = (J_PER_TEC + WT - 1) // WT
NPAIR = (NWIN + 1) // 2
IDX_BLK = 4096
ROWCAP = 128


@jax.jit
def _embed_lookup(target, table):
    info = plsc.get_sparse_core_info()
    nc, ns = info.num_cores, info.num_subcores
    nw = nc * ns

    mesh = plsc.VectorSubcoreMesh(core_axis_name="c", subcore_axis_name="s")

    @functools.partial(
        pl.kernel,
        mesh=mesh,
        out_type=jax.ShapeDtypeStruct((BATCH, TILE_W), jnp.float32),
        compiler_params=pltpu.CompilerParams(needs_layout_passes=False),
        scratch_types=[
            pltpu.VMEM((IDX_BLK + LANES,), jnp.int32),
            pltpu.VMEM((BATCH + LANES,), jnp.int32),
            pltpu.VMEM((BATCH + LANES,), jnp.int32),
            pltpu.VMEM((2 * LANES,), jnp.int32),
            pltpu.VMEM((2 * LANES,), jnp.int32),
            pltpu.VMEM((2, WT, EMBED_SIZE, TILE_W), jnp.float32),
            pltpu.VMEM((EMBED_SIZE, TILE_W), jnp.float32),
            pltpu.VMEM((ROWCAP, TILE_W), jnp.float32),
            pltpu.VMEM((ROWCAP + LANES,), jnp.int32),
            pltpu.SMEM((8,), jnp.int32),
            pltpu.SMEM((NWIN + 2,), jnp.int32),
            pltpu.SemaphoreType.DMA,
            pltpu.SemaphoreType.DMA,
            pltpu.SemaphoreType.DMA,
        ],
    )
    def gather_kernel(idx_hbm, table_hbm, edge_hbm, out_hbm, idxbuf, hit_r,
                      hit_b, whr, whb, slots, edgeslot, rowsbuf, bidx,
                      sc, wcnt, semA, semB, sem_scat):
        wid = lax.axis_index("s") * nc + lax.axis_index("c")
        lo = wid * J_PER_TEC
        hi = jnp.minimum(lo + J_PER_TEC, NUM_J)
        pltpu.sync_copy(edge_hbm, edgeslot)
        sems = (semA, semB)
        lane = lax.iota(jnp.int32, LANES)
        lane0 = lane == 0

        sc[0] = 0
        for blk in range(BATCH // IDX_BLK):
            pltpu.sync_copy(idx_hbm.at[pl.ds(blk * IDX_BLK, IDX_BLK)],
                            idxbuf.at[pl.ds(0, IDX_BLK)])

            def scan_body(k, _, blk=blk):
                v = idxbuf[pl.ds(k * LANES, LANES)]
                j16 = lax.shift_right_logical(v, 7)
                m = jnp.logical_and(j16 >= lo, j16 < hi)
                nh = sc[0]
                plsc.store_compressed(hit_r.at[pl.ds(nh, LANES)], v, mask=m)
                b16 = lane + (blk * IDX_BLK + k * LANES)
                plsc.store_compressed(hit_b.at[pl.ds(nh, LANES)], b16, mask=m)
                sc[0] = nh + plsc.all_reduce_population_count(m)[0]
                return 0

            lax.fori_loop(0, IDX_BLK // LANES, scan_body, 0)

        nh = sc[0]

        for t in range(NWIN):
            wcnt[t] = 0

        def cnt_body(i, _):
            r = hit_r[pl.ds(i, LANES)][0]
            w = lax.shift_right_logical(lax.shift_right_logical(r, 7) - lo, 2)
            wcnt[w] = wcnt[w] + 1
            return 0

        lax.fori_loop(0, nh, cnt_body, 0)

        def fetch_win(w, p):
            for k in range(WT):
                jf = jnp.minimum(lo + w * WT + k, LAST_J - 1)
                pltpu.make_async_copy(
                    table_hbm.at[pl.ds(0, EMBED_SIZE),
                                 pl.ds(jf * TILE_W, TILE_W)],
                    slots.at[p, k],
                    sems[p],
                ).start()

        def drain_win(p):
            for k in range(WT):
                pltpu.make_async_copy(
                    table_hbm.at[pl.ds(0, EMBED_SIZE), pl.ds(0, TILE_W)],
                    slots.at[p, k],
                    sems[p],
                ).wait()

        def flush():
            pltpu.async_copy(rowsbuf, out_hbm.at[bidx.at[pl.ds(0, ROWCAP)]],
                             sem_scat).wait()
            sc[1] = 0

        def emit_row(r, b, src_gather):
            l_vec = jnp.full((LANES,), r & (TILE_W - 1), dtype=jnp.int32)
            nrow = sc[1]
            for c0 in range(0, EMBED_SIZE, LANES):
                c_vec = lane + c0
                rowsbuf[nrow, pl.ds(c0, LANES)] = src_gather(c_vec, l_vec)
            plsc.store_scatter(bidx, [jnp.full((LANES,), nrow)],
                               jnp.full((LANES,), b), mask=lane0)
            sc[1] = nrow + 1

            @pl.when(nrow + 1 == ROWCAP)
            def _():
                flush()

        def process_win(w, p):
            wlo = lo + w * WT
            nchunks = lax.shift_right_logical(nh + LANES - 1, 4)

            def chunk_body(k, _):
                rv = hit_r[pl.ds(k * LANES, LANES)]
                bv = hit_b[pl.ds(k * LANES, LANES)]
                jv = lax.shift_right_logical(rv, 7)
                valid = (lane + k * LANES) < nh
                m = jnp.logical_and(
                    jnp.logical_and(jv >= wlo, jv < wlo + WT), valid)
                cw = plsc.all_reduce_population_count(m)[0]
                plsc.store_compressed(whr.at[pl.ds(0, LANES)], rv, mask=m)
                plsc.store_compressed(whb.at[pl.ds(0, LANES)], bv, mask=m)

                def hit_body(t, _):
                    r = whr[pl.ds(t, LANES)][0]
                    b = whb[pl.ds(t, LANES)][0]
                    j = lax.shift_right_logical(r, 7)
                    is_edge = j == LAST_J

                    @pl.when(jnp.logical_not(is_edge))
                    def _():
                        k_in_w = jnp.full((LANES,), j - wlo)
                        emit_row(r, b, lambda c, l: plsc.load_gather(
                            slots.at[p], [k_in_w, c, l]))

                    @pl.when(is_edge)
                    def _():
                        emit_row(r, b, lambda c, l: plsc.load_gather(
                            edgeslot, [c, l]))

                    return 0

                lax.fori_loop(0, cw, hit_body, 0)
                return 0

            lax.fori_loop(0, nchunks, chunk_body, 0)

        sc[1] = 0

        @pl.when(wcnt[0] > 0)
        def _():
            fetch_win(0, 0)

        def wpair(o, _):
            for par in range(2):
                w = o * 2 + par

                @pl.when(jnp.logical_and(w + 1 < NWIN, wcnt[w + 1] > 0))
                def _(w=w, par=par):
                    fetch_win(w + 1, 1 - par)

                @pl.when(wcnt[w] > 0)
                def _(w=w, par=par):
                    drain_win(par)
                    process_win(w, par)

            return 0

        lax.fori_loop(0, NPAIR, wpair, 0)

        nrow = sc[1]

        @pl.when(nrow > 0)
        def _():
            b0 = bidx[pl.ds(0, LANES)][0]

            def pad_body(t, _):
                for c0 in range(0, EMBED_SIZE, LANES):
                    rowsbuf[t, pl.ds(c0, LANES)] = rowsbuf[0, pl.ds(c0, LANES)]
                plsc.store_scatter(bidx, [jnp.full((LANES,), t)],
                                   jnp.full((LANES,), b0), mask=lane0)
                return 0

            lax.fori_loop(nrow, ROWCAP, pad_body, 0)
            flush()

    table_t = table.T
    edge_p = jnp.pad(
        table[NUM_EMBEDDINGS - LAST_W:, :].T,
        ((0, 0), (0, TILE_W - LAST_W)),
    )
    out_p = gather_kernel(target.astype(jnp.int32), table_t, edge_p)
    return out_p[:, :EMBED_SIZE]


def kernel(target, table):
    return _embed_lookup(target, table)


# speedup vs baseline: 1.0438x; 1.0438x over previous
import functools

import jax
import jax.numpy as jnp
from jax import lax
from jax.experimental import pallas as pl
from jax.experimental.pallas import tpu as pltpu
from jax.experimental.pallas import tpu_sc as plsc

NUM_EMBEDDINGS = 1000000
EMBED_SIZE = 64
BATCH = 16384
LANES = 16
TILE_W = 128
LAST_J = NUM_EMBEDDINGS // TILE_W
LAST_W = NUM_EMBEDDINGS - LAST_J * TILE_W
NUM_J = LAST_J + 1
J_PER_TEC = 245
WT = 4
NWIN = (J_PER_TEC + WT - 1) // WT
NPAIR = (NWIN + 1) // 2
IDX_BLK = 4096
ROWCAP = 64


@jax.jit
def _embed_lookup(target, table):
    info = plsc.get_sparse_core_info()
    nc, ns = info.num_cores, info.num_subcores
    nw = nc * ns

    mesh = plsc.VectorSubcoreMesh(core_axis_name="c", subcore_axis_name="s")

    @functools.partial(
        pl.kernel,
        mesh=mesh,
        out_type=jax.ShapeDtypeStruct((BATCH, TILE_W), jnp.float32),
        compiler_params=pltpu.CompilerParams(needs_layout_passes=False),
        scratch_types=[
            pltpu.VMEM((IDX_BLK + LANES,), jnp.int32),
            pltpu.VMEM((BATCH + LANES,), jnp.int32),
            pltpu.VMEM((BATCH + LANES,), jnp.int32),
            pltpu.VMEM((2 * LANES,), jnp.int32),
            pltpu.VMEM((2 * LANES,), jnp.int32),
            pltpu.VMEM((2, WT, EMBED_SIZE, TILE_W), jnp.float32),
            pltpu.VMEM((EMBED_SIZE, TILE_W), jnp.float32),
            pltpu.VMEM((ROWCAP, TILE_W), jnp.float32),
            pltpu.VMEM((ROWCAP + LANES,), jnp.int32),
            pltpu.SMEM((8,), jnp.int32),
            pltpu.SemaphoreType.DMA,
            pltpu.SemaphoreType.DMA,
            pltpu.SemaphoreType.DMA,
        ],
    )
    def gather_kernel(idx_hbm, table_hbm, edge_hbm, out_hbm, idxbuf, hit_r,
                      hit_b, whr, whb, slots, edgeslot, rowsbuf, bidx,
                      sc, semA, semB, sem_scat):
        wid = lax.axis_index("s") * nc + lax.axis_index("c")
        lo = wid * J_PER_TEC
        hi = jnp.minimum(lo + J_PER_TEC, NUM_J)
        sems = (semA, semB)
        lane = lax.iota(jnp.int32, LANES)
        lane0 = lane == 0

        def fetch_win(w, p):
            for k in range(WT):
                jf = jnp.minimum(lo + w * WT + k, LAST_J - 1)
                pltpu.make_async_copy(
                    table_hbm.at[pl.ds(0, EMBED_SIZE),
                                 pl.ds(jf * TILE_W, TILE_W)],
                    slots.at[p, k],
                    sems[p],
                ).start()

        fetch_win(0, 0)
        fetch_win(1, 1)
        pltpu.sync_copy(edge_hbm, edgeslot)

        sc[0] = 0
        for blk in range(BATCH // IDX_BLK):
            pltpu.sync_copy(idx_hbm.at[pl.ds(blk * IDX_BLK, IDX_BLK)],
                            idxbuf.at[pl.ds(0, IDX_BLK)])

            def scan_body(k, _, blk=blk):
                v = idxbuf[pl.ds(k * LANES, LANES)]
                j16 = lax.shift_right_logical(v, 7)
                m = jnp.logical_and(j16 >= lo, j16 < hi)
                nh = sc[0]
                plsc.store_compressed(hit_r.at[pl.ds(nh, LANES)], v, mask=m)
                b16 = lane + (blk * IDX_BLK + k * LANES)
                plsc.store_compressed(hit_b.at[pl.ds(nh, LANES)], b16, mask=m)
                sc[0] = nh + plsc.all_reduce_population_count(m)[0]
                return 0

            lax.fori_loop(0, IDX_BLK // LANES, scan_body, 0)

        nh = sc[0]

        def drain_win(p):
            for k in range(WT):
                pltpu.make_async_copy(
                    table_hbm.at[pl.ds(0, EMBED_SIZE), pl.ds(0, TILE_W)],
                    slots.at[p, k],
                    sems[p],
                ).wait()

        def flush():
            pltpu.async_copy(rowsbuf, out_hbm.at[bidx.at[pl.ds(0, ROWCAP)]],
                             sem_scat).wait()
            sc[1] = 0

        def emit_row(r, b, src_gather):
            l_vec = jnp.full((LANES,), r & (TILE_W - 1), dtype=jnp.int32)
            nrow = sc[1]
            for c0 in range(0, EMBED_SIZE, LANES):
                c_vec = lane + c0
                rowsbuf[nrow, pl.ds(c0, LANES)] = src_gather(c_vec, l_vec)
            plsc.store_scatter(bidx, [jnp.full((LANES,), nrow)],
                               jnp.full((LANES,), b), mask=lane0)
            sc[1] = nrow + 1

            @pl.when(nrow + 1 == ROWCAP)
            def _():
                flush()

        def process_win(w, p):
            wlo = lo + w * WT
            nchunks = lax.shift_right_logical(nh + LANES - 1, 4)

            def chunk_body(k, _):
                rv = hit_r[pl.ds(k * LANES, LANES)]
                bv = hit_b[pl.ds(k * LANES, LANES)]
                jv = lax.shift_right_logical(rv, 7)
                valid = (lane + k * LANES) < nh
                m = jnp.logical_and(
                    jnp.logical_and(jv >= wlo, jv < wlo + WT), valid)
                cw = plsc.all_reduce_population_count(m)[0]
                plsc.store_compressed(whr.at[pl.ds(0, LANES)], rv, mask=m)
                plsc.store_compressed(whb.at[pl.ds(0, LANES)], bv, mask=m)

                def hit_body(t, _):
                    r = whr[pl.ds(t, LANES)][0]
                    b = whb[pl.ds(t, LANES)][0]
                    j = lax.shift_right_logical(r, 7)
                    is_edge = j == LAST_J

                    @pl.when(jnp.logical_not(is_edge))
                    def _():
                        k_in_w = jnp.full((LANES,), j - wlo)
                        emit_row(r, b, lambda c, l: plsc.load_gather(
                            slots.at[p], [k_in_w, c, l]))

                    @pl.when(is_edge)
                    def _():
                        emit_row(r, b, lambda c, l: plsc.load_gather(
                            edgeslot, [c, l]))

                    return 0

                lax.fori_loop(0, cw, hit_body, 0)
                return 0

            lax.fori_loop(0, nchunks, chunk_body, 0)

        sc[1] = 0

        def wpair(o, _):
            for par in range(2):
                w = o * 2 + par
                drain_win(par)
                process_win(w, par)

                @pl.when(w + 2 < NWIN)
                def _(w=w, par=par):
                    fetch_win(w + 2, par)

            return 0

        lax.fori_loop(0, NPAIR, wpair, 0)

        nrow = sc[1]

        @pl.when(nrow > 0)
        def _():
            b0 = bidx[pl.ds(0, LANES)][0]

            def pad_body(t, _):
                for c0 in range(0, EMBED_SIZE, LANES):
                    rowsbuf[t, pl.ds(c0, LANES)] = rowsbuf[0, pl.ds(c0, LANES)]
                plsc.store_scatter(bidx, [jnp.full((LANES,), t)],
                                   jnp.full((LANES,), b0), mask=lane0)
                return 0

            lax.fori_loop(nrow, ROWCAP, pad_body, 0)
            flush()

    table_t = table.T
    edge_p = jnp.pad(
        table[NUM_EMBEDDINGS - LAST_W:, :].T,
        ((0, 0), (0, TILE_W - LAST_W)),
    )
    out_p = gather_kernel(target.astype(jnp.int32), table_t, edge_p)
    return out_p[:, :EMBED_SIZE]


def kernel(target, table):
    return _embed_lookup(target, table)
